# baseline (speedup 1.0000x reference)
import functools

import jax
import jax.numpy as jnp
from jax import lax
from jax.experimental import pallas as pl
from jax.experimental.pallas import tpu as pltpu
from jax.experimental.pallas import tpu_sc as plsc

_NTOKEN = 100000
_NINP = 64
_BATCH = 16384
_HIST = 50

_B = _BATCH * _HIST
_NC = 2
_NS = 16
_NW = _NC * _NS
_BPW = _B // _NW
_C = 128
_K = 4
_RPB = _K * _C
_NBLK = _BPW // _RPB
_NK2 = _NBLK // 2
_NCH = _B // _C


def _make_gather():
    mesh = plsc.VectorSubcoreMesh(core_axis_name="c", subcore_axis_name="s")

    @functools.partial(
        pl.kernel,
        mesh=mesh,
        out_type=jax.ShapeDtypeStruct((_B, _NINP), jnp.float32),
        scratch_types=[
            pltpu.VMEM((_NBLK, _RPB), jnp.int32),
            pltpu.VMEM((2, _RPB, _NINP), jnp.float32),
            pltpu.SemaphoreType.DMA,
            pltpu.SemaphoreType.DMA,
            pltpu.SemaphoreType.DMA,
            pltpu.SemaphoreType.DMA,
        ],
        compiler_params=pltpu.CompilerParams(use_tc_tiling_on_sc=False),
    )
    def gather(table_hbm, idx_hbm, out_hbm, idx_v, rows_v, gs0, gs1, ws0, ws1):
        wid = lax.axis_index("s") * _NC + lax.axis_index("c")
        rbase = wid * _BPW
        pltpu.sync_copy(idx_hbm.at[wid], idx_v)

        def fire_gather(blk, half, sem):
            pltpu.async_copy(table_hbm.at[idx_v.at[blk]], rows_v.at[half], sem)

        def drain_gather(half, sem):
            pltpu.make_async_copy(
                table_hbm.at[idx_v.at[0]], rows_v.at[half], sem
            ).wait()

        def fire_write(blk, half, sem):
            pltpu.async_copy(
                rows_v.at[half],
                out_hbm.at[pl.ds(rbase + blk * _RPB, _RPB)],
                sem,
            )

        def drain_write(half, sem):
            pltpu.make_async_copy(
                rows_v.at[half], out_hbm.at[pl.ds(rbase, _RPB)], sem
            ).wait()

        fire_gather(0, 0, gs0)

        def body(k, carry):
            b0 = 2 * k

            @pl.when(k > 0)
            def _():
                drain_write(1, ws1)

            fire_gather(b0 + 1, 1, gs1)
            drain_gather(0, gs0)
            fire_write(b0, 0, ws0)
            drain_gather(1, gs1)
            fire_write(b0 + 1, 1, ws1)

            @pl.when(k < _NK2 - 1)
            def _():
                drain_write(0, ws0)
                fire_gather(b0 + 2, 0, gs0)

            return carry

        lax.fori_loop(0, _NK2, body, 0)
        drain_write(0, ws0)
        drain_write(1, ws1)

    return gather


_gather = _make_gather()


def kernel(input, word_embed_weight):
    idx = input.reshape(_NW, _NBLK, _RPB)
    out = _gather(word_embed_weight, idx)
    return out.reshape(_BATCH, _HIST, _NINP)


# speedup vs baseline: 1.0107x; 1.0107x over previous
import functools

import jax
import jax.numpy as jnp
from jax import lax
from jax.experimental import pallas as pl
from jax.experimental.pallas import tpu as pltpu
from jax.experimental.pallas import tpu_sc as plsc

_NTOKEN = 100000
_NINP = 64
_BATCH = 16384
_HIST = 50

_B = _BATCH * _HIST
_NC = 2
_NS = 16
_NW = _NC * _NS
_BPW = _B // _NW
_RPB = 256
_NBLK = _BPW // _RPB
_NR = 6
_G = 3
_NSTEP = _NBLK + _G + _NR
_NK = -(-_NSTEP // _NR)


def _make_gather():
    mesh = plsc.VectorSubcoreMesh(core_axis_name="c", subcore_axis_name="s")

    @functools.partial(
        pl.kernel,
        mesh=mesh,
        out_type=jax.ShapeDtypeStruct((_B, _NINP), jnp.float32),
        scratch_types=[
            pltpu.VMEM((_NBLK, _RPB), jnp.int32),
            pltpu.VMEM((_NR, _RPB, _NINP), jnp.float32),
        ]
        + [pltpu.SemaphoreType.DMA] * (2 * _NR),
        compiler_params=pltpu.CompilerParams(use_tc_tiling_on_sc=False),
    )
    def gather(table_hbm, idx_hbm, out_hbm, idx_v, rows_v, *sems):
        gsem = sems[:_NR]
        wsem = sems[_NR:]
        wid = lax.axis_index("s") * _NC + lax.axis_index("c")
        rbase = wid * _BPW
        pltpu.sync_copy(idx_hbm.at[wid], idx_v)

        def fire_gather(blk, r):
            pltpu.async_copy(table_hbm.at[idx_v.at[blk]], rows_v.at[r], gsem[r])

        def drain_gather(r):
            pltpu.make_async_copy(
                table_hbm.at[idx_v.at[0]], rows_v.at[r], gsem[r]
            ).wait()

        def fire_write(blk, r):
            pltpu.async_copy(
                rows_v.at[r], out_hbm.at[pl.ds(rbase + blk * _RPB, _RPB)], wsem[r]
            )

        def drain_write(r):
            pltpu.make_async_copy(
                rows_v.at[r], out_hbm.at[pl.ds(rbase, _RPB)], wsem[r]
            ).wait()

        def body(k, carry):
            for r in range(_NR):
                j = k * _NR + r

                @pl.when(jnp.logical_and(j >= _NR, j - _NR < _NBLK))
                def _():
                    drain_write(r)

                @pl.when(j < _NBLK)
                def _():
                    fire_gather(j, r)

                jw = j - _G
                rw = (r - _G) % _NR

                @pl.when(jnp.logical_and(jw >= 0, jw < _NBLK))
                def _():
                    drain_gather(rw)
                    fire_write(jw, rw)
            return carry

        lax.fori_loop(0, _NK, body, 0)

    return gather


_gather = _make_gather()


def kernel(input, word_embed_weight):
    idx = input.reshape(_NW, _NBLK, _RPB)
    out = _gather(word_embed_weight, idx)
    return out.reshape(_BATCH, _HIST, _NINP)
